# baseline (speedup 1.0000x reference)
import jax
import jax.numpy as jnp
from jax import lax
from jax.experimental import pallas as pl
from jax.experimental.pallas import tpu as pltpu
from jax.experimental.pallas import tpu_sc as plsc

FEATURE_DIM = 100000
EMBEDDING_DIM = 64

NUM_CORES = 2
NUM_SUBCORES = 16
NUM_WORKERS = NUM_CORES * NUM_SUBCORES

BATCH = 4096
SEQ = 50
TOTAL = BATCH * SEQ
PER_WORKER = TOTAL // NUM_WORKERS
NBUF = 3
CHUNK = 640
NUM_CHUNKS = PER_WORKER // CHUNK


def _gather_body(idx_hbm, table_hbm, out_hbm,
                 idx_v, rows0, rows1, rows2, g0, g1, g2, s0, s1, s2):
    wid = lax.axis_index("s") * NUM_CORES + lax.axis_index("c")
    base = wid * PER_WORKER

    rows_b = (rows0, rows1, rows2)
    gsem = (g0, g1, g2)
    ssem = (s0, s1, s2)
    gathers = [None] * NBUF
    stores = [None] * NBUF

    pltpu.sync_copy(idx_hbm.at[pl.ds(base, PER_WORKER)], idx_v)

    for i in range(NUM_CHUNKS):
        b = i % NBUF
        if i >= NBUF:
            stores[b].wait()
        gathers[b] = pltpu.async_copy(
            table_hbm.at[idx_v.at[pl.ds(i * CHUNK, CHUNK)]], rows_b[b],
            gsem[b])
        if i >= 1:
            pb = (i - 1) % NBUF
            gathers[pb].wait()
            stores[pb] = pltpu.async_copy(
                rows_b[pb],
                out_hbm.at[pl.ds(base + (i - 1) * CHUNK, CHUNK)],
                ssem[pb])

    last = (NUM_CHUNKS - 1) % NBUF
    gathers[last].wait()
    stores[last] = pltpu.async_copy(
        rows_b[last],
        out_hbm.at[pl.ds(base + (NUM_CHUNKS - 1) * CHUNK, CHUNK)],
        ssem[last])
    for b in range(NBUF):
        if b != last:
            stores[b].wait()
    stores[last].wait()


@jax.jit
def _gather(idx, table):
    mesh = plsc.VectorSubcoreMesh(core_axis_name="c", subcore_axis_name="s",
                                  num_cores=NUM_CORES,
                                  num_subcores=NUM_SUBCORES)
    return pl.kernel(
        _gather_body,
        out_type=jax.ShapeDtypeStruct((TOTAL, EMBEDDING_DIM), jnp.float32),
        mesh=mesh,
        scratch_types=[
            pltpu.VMEM((PER_WORKER,), jnp.int32),
            pltpu.VMEM((CHUNK, EMBEDDING_DIM), jnp.float32),
            pltpu.VMEM((CHUNK, EMBEDDING_DIM), jnp.float32),
            pltpu.VMEM((CHUNK, EMBEDDING_DIM), jnp.float32),
            pltpu.SemaphoreType.DMA,
            pltpu.SemaphoreType.DMA,
            pltpu.SemaphoreType.DMA,
            pltpu.SemaphoreType.DMA,
            pltpu.SemaphoreType.DMA,
            pltpu.SemaphoreType.DMA,
        ],
        compiler_params=pltpu.CompilerParams(use_tc_tiling_on_sc=False),
    )(idx, table)


def kernel(x, table):
    idx = x.reshape(-1).astype(jnp.int32)
    out2d = _gather(idx, table)
    return out2d.reshape(BATCH, SEQ, EMBEDDING_DIM)


# speedup vs baseline: 1.0072x; 1.0072x over previous
import jax
import jax.numpy as jnp
from jax import lax
from jax.experimental import pallas as pl
from jax.experimental.pallas import tpu as pltpu
from jax.experimental.pallas import tpu_sc as plsc

FEATURE_DIM = 100000
EMBEDDING_DIM = 64

NUM_CORES = 2
NUM_SUBCORES = 16
NUM_WORKERS = NUM_CORES * NUM_SUBCORES

BATCH = 4096
SEQ = 50
TOTAL = BATCH * SEQ
PER_WORKER = TOTAL // NUM_WORKERS
NBUF = 2
CHUNK = 800
NUM_CHUNKS = PER_WORKER // CHUNK


def _gather_body(idx_hbm, table_hbm, out_hbm,
                 idx_v, rows0, rows1, g0, g1, s0, s1):
    wid = lax.axis_index("s") * NUM_CORES + lax.axis_index("c")
    base = wid * PER_WORKER

    rows_b = (rows0, rows1)
    gsem = (g0, g1)
    ssem = (s0, s1)
    gathers = [None] * NBUF
    stores = [None] * NBUF

    pltpu.sync_copy(idx_hbm.at[pl.ds(base, PER_WORKER)], idx_v)

    for i in range(NUM_CHUNKS):
        b = i % NBUF
        if i >= NBUF:
            stores[b].wait()
        gathers[b] = pltpu.async_copy(
            table_hbm.at[idx_v.at[pl.ds(i * CHUNK, CHUNK)]], rows_b[b],
            gsem[b])
        if i >= 1:
            pb = (i - 1) % NBUF
            gathers[pb].wait()
            stores[pb] = pltpu.async_copy(
                rows_b[pb],
                out_hbm.at[pl.ds(base + (i - 1) * CHUNK, CHUNK)],
                ssem[pb])

    last = (NUM_CHUNKS - 1) % NBUF
    gathers[last].wait()
    stores[last] = pltpu.async_copy(
        rows_b[last],
        out_hbm.at[pl.ds(base + (NUM_CHUNKS - 1) * CHUNK, CHUNK)],
        ssem[last])
    for b in range(NBUF):
        if b != last:
            stores[b].wait()
    stores[last].wait()


@jax.jit
def _gather(idx, table):
    mesh = plsc.VectorSubcoreMesh(core_axis_name="c", subcore_axis_name="s",
                                  num_cores=NUM_CORES,
                                  num_subcores=NUM_SUBCORES)
    return pl.kernel(
        _gather_body,
        out_type=jax.ShapeDtypeStruct((TOTAL, EMBEDDING_DIM), jnp.float32),
        mesh=mesh,
        scratch_types=[
            pltpu.VMEM((PER_WORKER,), jnp.int32),
            pltpu.VMEM((CHUNK, EMBEDDING_DIM), jnp.float32),
            pltpu.VMEM((CHUNK, EMBEDDING_DIM), jnp.float32),
            pltpu.SemaphoreType.DMA,
            pltpu.SemaphoreType.DMA,
            pltpu.SemaphoreType.DMA,
            pltpu.SemaphoreType.DMA,
        ],
        compiler_params=pltpu.CompilerParams(use_tc_tiling_on_sc=False),
    )(idx, table)


def kernel(x, table):
    idx = x.reshape(-1).astype(jnp.int32)
    out2d = _gather(idx, table)
    return out2d.reshape(BATCH, SEQ, EMBEDDING_DIM)
